# Initial kernel scaffold; baseline (speedup 1.0000x reference)
import jax
import jax.numpy as jnp
from jax.experimental import pallas as pl


def kernel(x, A, W1_l, b1, W1_r, W2_l, b2, W2_r):
    raise NotImplementedError("write your pallas kernel here")


# speedup vs baseline: 1.5113x; 1.5113x over previous
import jax
import jax.numpy as jnp
from jax import lax
from jax.experimental import pallas as pl
from jax.experimental.pallas import tpu as pltpu

_B, _N = 4, 2048
_IN_C, _HID, _OUT_C = 128, 128, 64


def _sage_body(a_ref, x_ref, w1l_ref, w1r_ref, b1_ref, w2l_ref, w2r_ref,
               b2_ref, logp_ref, out_ref):
    a = a_ref[0]
    mask = (a != 0).astype(jnp.bfloat16)

    deg = jnp.sum((a != 0).astype(jnp.int32), axis=0)
    deg_f = deg.astype(jnp.float32).reshape(_N, 1)
    inv = jnp.where(deg_f > 0.0, 1.0 / jnp.maximum(deg_f, 1.0), 0.0)

    xb = x_ref[0]
    x16 = xb.astype(jnp.bfloat16)

    agg1 = lax.dot_general(mask, x16, (((0,), (0,)), ((), ())),
                           preferred_element_type=jnp.float32)
    aggn1 = (agg1 * inv).astype(jnp.bfloat16)
    h1 = (lax.dot_general(aggn1, w1l_ref[...], (((1,), (1,)), ((), ())),
                          preferred_element_type=jnp.float32)
          + lax.dot_general(x16, w1r_ref[...], (((1,), (1,)), ((), ())),
                            preferred_element_type=jnp.float32)
          + b1_ref[...])
    h1 = jnp.maximum(h1, 0.0)
    h16 = h1.astype(jnp.bfloat16)

    agg2 = lax.dot_general(mask, h16, (((0,), (0,)), ((), ())),
                           preferred_element_type=jnp.float32)
    aggn2 = (agg2 * inv).astype(jnp.bfloat16)
    out = (lax.dot_general(aggn2, w2l_ref[...], (((1,), (1,)), ((), ())),
                           preferred_element_type=jnp.float32)
           + lax.dot_general(h16, w2r_ref[...], (((1,), (1,)), ((), ())),
                             preferred_element_type=jnp.float32)
           + b2_ref[...])

    m = jnp.max(out, axis=-1, keepdims=True)
    lse = jnp.log(jnp.sum(jnp.exp(out - m), axis=-1, keepdims=True)) + m
    logp_ref[0] = out - lse
    out_ref[0] = out


def kernel(x, A, W1_l, b1, W1_r, W2_l, b2, W2_r):
    w16 = jnp.bfloat16
    grid = (_B,)
    logp, out = pl.pallas_call(
        _sage_body,
        grid=grid,
        in_specs=[
            pl.BlockSpec((1, _N, _N), lambda b: (b, 0, 0)),
            pl.BlockSpec((1, _N, _IN_C), lambda b: (b, 0, 0)),
            pl.BlockSpec((_HID, _IN_C), lambda b: (0, 0)),
            pl.BlockSpec((_HID, _IN_C), lambda b: (0, 0)),
            pl.BlockSpec((1, _HID), lambda b: (0, 0)),
            pl.BlockSpec((_OUT_C, _HID), lambda b: (0, 0)),
            pl.BlockSpec((_OUT_C, _HID), lambda b: (0, 0)),
            pl.BlockSpec((1, _OUT_C), lambda b: (0, 0)),
        ],
        out_specs=[
            pl.BlockSpec((1, _N, _OUT_C), lambda b: (b, 0, 0)),
            pl.BlockSpec((1, _N, _OUT_C), lambda b: (b, 0, 0)),
        ],
        out_shape=[
            jax.ShapeDtypeStruct((_B, _N, _OUT_C), jnp.float32),
            jax.ShapeDtypeStruct((_B, _N, _OUT_C), jnp.float32),
        ],
        compiler_params=pltpu.CompilerParams(
            dimension_semantics=("arbitrary",),
            vmem_limit_bytes=128 * 1024 * 1024,
        ),
    )(A, x,
      W1_l.astype(w16), W1_r.astype(w16), b1.reshape(1, _HID),
      W2_l.astype(w16), W2_r.astype(w16), b2.reshape(1, _OUT_C))
    return (logp, out)


# speedup vs baseline: 1.8960x; 1.2546x over previous
import jax
import jax.numpy as jnp
from jax import lax
from jax.experimental import pallas as pl
from jax.experimental.pallas import tpu as pltpu

_B, _N = 4, 2048
_IN_C, _HID, _OUT_C = 128, 128, 64


def _sage_body(a_ref, xt_ref, w1_ref, b1_ref, w2_ref, b2_ref,
               logp_ref, out_ref):
    a = a_ref[0]
    mask = (a != 0).astype(jnp.bfloat16)

    ones = jnp.ones((8, _N), dtype=jnp.bfloat16)
    deg = lax.dot_general(ones, mask, (((1,), (0,)), ((), ())),
                          preferred_element_type=jnp.float32)[0:1]
    inv = jnp.where(deg > 0.0, 1.0 / jnp.maximum(deg, 1.0), 0.0)

    xt = xt_ref[0]

    agg1 = lax.dot_general(xt, mask, (((1,), (0,)), ((), ())),
                           preferred_element_type=jnp.float32)
    cat1 = jnp.concatenate([(agg1 * inv).astype(jnp.bfloat16), xt], axis=0)
    h1 = lax.dot_general(w1_ref[...], cat1, (((1,), (0,)), ((), ())),
                         preferred_element_type=jnp.float32) + b1_ref[...]
    h1t = jnp.maximum(h1, 0.0).astype(jnp.bfloat16)

    agg2 = lax.dot_general(h1t, mask, (((1,), (0,)), ((), ())),
                           preferred_element_type=jnp.float32)
    cat2 = jnp.concatenate([(agg2 * inv).astype(jnp.bfloat16), h1t], axis=0)
    out = lax.dot_general(cat2, w2_ref[...], (((0,), (1,)), ((), ())),
                          preferred_element_type=jnp.float32) + b2_ref[...]

    m = jnp.max(out, axis=-1, keepdims=True)
    lse = jnp.log(jnp.sum(jnp.exp(out - m), axis=-1, keepdims=True)) + m
    logp_ref[0] = out - lse
    out_ref[0] = out


def kernel(x, A, W1_l, b1, W1_r, W2_l, b2, W2_r):
    bf = jnp.bfloat16
    xt = jnp.transpose(x, (0, 2, 1)).astype(bf)
    w1 = jnp.concatenate([W1_l, W1_r], axis=1).astype(bf)
    w2 = jnp.concatenate([W2_l, W2_r], axis=1).astype(bf)
    logp, out = pl.pallas_call(
        _sage_body,
        grid=(_B,),
        in_specs=[
            pl.BlockSpec((1, _N, _N), lambda b: (b, 0, 0)),
            pl.BlockSpec((1, _IN_C, _N), lambda b: (b, 0, 0)),
            pl.BlockSpec((_HID, 2 * _IN_C), lambda b: (0, 0)),
            pl.BlockSpec((_HID, 1), lambda b: (0, 0)),
            pl.BlockSpec((_OUT_C, 2 * _HID), lambda b: (0, 0)),
            pl.BlockSpec((1, _OUT_C), lambda b: (0, 0)),
        ],
        out_specs=[
            pl.BlockSpec((1, _N, _OUT_C), lambda b: (b, 0, 0)),
            pl.BlockSpec((1, _N, _OUT_C), lambda b: (b, 0, 0)),
        ],
        out_shape=[
            jax.ShapeDtypeStruct((_B, _N, _OUT_C), jnp.float32),
            jax.ShapeDtypeStruct((_B, _N, _OUT_C), jnp.float32),
        ],
        compiler_params=pltpu.CompilerParams(
            dimension_semantics=("arbitrary",),
            vmem_limit_bytes=128 * 1024 * 1024,
        ),
    )(A, xt, w1, b1.reshape(_HID, 1), w2, b2.reshape(1, _OUT_C))
    return (logp, out)
